# baseline (device time: 403645 ns/iter reference)
import jax
import jax.numpy as jnp
from jax import lax
from jax.experimental import pallas as pl
from jax.experimental.pallas import tpu as pltpu

N_DEV = 4
S = 2048
H = 8
DH = 128
D = 1024
BLK = 64
SCALE = 0.08838834764831843
NEG = -1e9
QT = 256
N_QT = S // QT
N_HQ = H * N_QT


def _fused(kvp, xb, wqb, wob):
    def body(kvp_ref, xb_ref, wq_ref, wo_ref, out_ref,
             kvall_ref, q_ref, kvbuf, acc_ref, m_ref, l_ref, ctx_ref,
             sT_ref, chunk_sem, send_sems, recv_sems):
        my = lax.axis_index("i")

        barrier_sem = pltpu.get_barrier_semaphore()
        for off in range(1, N_DEV):
            pl.semaphore_signal(
                barrier_sem, inc=1,
                device_id=(lax.rem(my + off, N_DEV),),
                device_id_type=pl.DeviceIdType.MESH,
            )
        pl.semaphore_wait(barrier_sem, N_DEV - 1)

        for off in range(1, N_DEV):
            @pl.when(my + off <= N_DEV - 1)
            def _send():
                rdma = pltpu.make_async_remote_copy(
                    src_ref=kvp_ref,
                    dst_ref=kvall_ref.at[my],
                    send_sem=send_sems.at[off - 1],
                    recv_sem=recv_sems.at[my],
                    device_id=(my + off,),
                    device_id_type=pl.DeviceIdType.MESH,
                )
                rdma.start()

        for g in range(H // 2):
            qp = jnp.dot(xb_ref[...], wq_ref[:, g * 2 * DH:(g + 1) * 2 * DH],
                         preferred_element_type=jnp.float32) * SCALE
            q_ref[2 * g] = qp[:, 0:DH].astype(jnp.bfloat16)
            q_ref[2 * g + 1] = qp[:, DH:2 * DH].astype(jnp.bfloat16)
        m_ref[...] = jnp.full((N_HQ, 1, QT), -1e30, jnp.float32)
        l_ref[...] = jnp.zeros((N_HQ, 1, QT), jnp.float32)
        acc_ref[...] = jnp.zeros((N_HQ, DH, QT), jnp.float32)

        def flash_update(idx, vh, sT):
            m_prev = m_ref[idx]
            m_new = jnp.maximum(
                m_prev, jnp.max(sT, axis=0, keepdims=True))
            alpha = jnp.exp(m_prev - m_new)
            p = jnp.exp(sT - m_new)
            l_ref[idx] = l_ref[idx] * alpha + jnp.sum(
                p, axis=0, keepdims=True)
            acc_ref[idx] = acc_ref[idx] * alpha + lax.dot_general(
                vh, p.astype(jnp.bfloat16), (((0,), (0,)), ((), ())),
                preferred_element_type=jnp.float32,
            )
            m_ref[idx] = m_new

        def qk_tile(idx):
            h = idx // N_QT
            qt = idx % N_QT
            qh = q_ref[h, pl.ds(qt * QT, QT), :]
            kh = kvbuf[h, :, 0:DH]
            return lax.dot_general(
                kh, qh, (((1,), (1,)), ((), ())),
                preferred_element_type=jnp.float32,
            ).astype(jnp.bfloat16)

        def run_full_chunk():
            sT_ref[0] = qk_tile(0)

            def hq_body(idx, dummy):
                nxt = jnp.minimum(idx + 1, N_HQ - 1)
                sT_ref[lax.rem(idx + 1, 2)] = qk_tile(nxt)
                h = idx // N_QT
                vh = kvbuf[h, :, DH:2 * DH]
                flash_update(idx, vh,
                             sT_ref[lax.rem(idx, 2)].astype(jnp.float32))
                return dummy

            lax.fori_loop(0, N_HQ, hq_body, 0)

        def make_diag_body(qt):
            rs = qt * QT
            kl = rs + QT

            def diag_body(h, dummy):
                qh = q_ref[h, rs:rs + QT, :]
                kh = kvbuf[h, 0:kl, 0:DH]
                vh = kvbuf[h, 0:kl, DH:2 * DH]
                sT = lax.dot_general(
                    kh, qh, (((1,), (1,)), ((), ())),
                    preferred_element_type=jnp.float32,
                )
                kb = lax.broadcasted_iota(jnp.int32, (kl, 1), 0) // BLK
                qb = (rs
                      + lax.broadcasted_iota(jnp.int32, (1, QT), 1)
                      ) // BLK
                sT = jnp.where(kb <= qb, sT, NEG)
                flash_update(h * N_QT + qt, vh, sT)
                return dummy
            return diag_body

        for k in range(N_DEV):
            c = my - k

            @pl.when(c >= 0)
            def _chunk():
                if k == 0:
                    cp = pltpu.make_async_copy(kvp_ref, kvbuf, chunk_sem)
                    cp.start()
                    cp.wait()
                    for qt in range(N_QT):
                        lax.fori_loop(0, H, make_diag_body(qt), 0)
                else:
                    pltpu.make_async_remote_copy(
                        src_ref=kvall_ref.at[c],
                        dst_ref=kvall_ref.at[c],
                        send_sem=send_sems.at[0],
                        recv_sem=recv_sems.at[c],
                        device_id=(my,),
                        device_id_type=pl.DeviceIdType.MESH,
                    ).wait_recv()
                    cp = pltpu.make_async_copy(
                        kvall_ref.at[c], kvbuf, chunk_sem)
                    cp.start()
                    cp.wait()
                    run_full_chunk()

        def ep_body(idx, dummy):
            h = idx // N_QT
            qt = idx % N_QT
            rs = qt * QT
            ctx_ref[pl.ds(h * DH, DH), pl.ds(rs, QT)] = (
                acc_ref[idx] / l_ref[idx]).astype(jnp.bfloat16)
            return dummy

        lax.fori_loop(0, N_HQ, ep_body, 0)
        for rb in range(4):
            rs = rb * (S // 4)
            out_ref[0, rs:rs + S // 4, :] = lax.dot_general(
                ctx_ref[:, rs:rs + S // 4], wo_ref[...],
                (((0,), (0,)), ((), ())),
                preferred_element_type=jnp.float32,
            )

        for off in range(1, N_DEV):
            @pl.when(my + off <= N_DEV - 1)
            def _drain():
                pltpu.make_async_remote_copy(
                    src_ref=kvp_ref,
                    dst_ref=kvall_ref.at[my],
                    send_sem=send_sems.at[off - 1],
                    recv_sem=recv_sems.at[0],
                    device_id=(my,),
                    device_id_type=pl.DeviceIdType.MESH,
                ).wait_send()

    out, _ = pl.pallas_call(
        body,
        out_shape=[
            jax.ShapeDtypeStruct((1, S, D), jnp.float32),
            jax.ShapeDtypeStruct((N_DEV, H, S, 2 * DH), jnp.bfloat16),
        ],
        in_specs=[
            pl.BlockSpec(memory_space=pltpu.MemorySpace.HBM),
            pl.BlockSpec(memory_space=pltpu.MemorySpace.VMEM),
            pl.BlockSpec(memory_space=pltpu.MemorySpace.VMEM),
            pl.BlockSpec(memory_space=pltpu.MemorySpace.VMEM),
        ],
        out_specs=[
            pl.BlockSpec(memory_space=pltpu.MemorySpace.VMEM),
            pl.BlockSpec(memory_space=pltpu.MemorySpace.HBM),
        ],
        scratch_shapes=[
            pltpu.VMEM((H, S, DH), jnp.bfloat16),
            pltpu.VMEM((H, S, 2 * DH), jnp.bfloat16),
            pltpu.VMEM((N_HQ, DH, QT), jnp.float32),
            pltpu.VMEM((N_HQ, 1, QT), jnp.float32),
            pltpu.VMEM((N_HQ, 1, QT), jnp.float32),
            pltpu.VMEM((H * DH, S), jnp.bfloat16),
            pltpu.VMEM((2, S, QT), jnp.bfloat16),
            pltpu.SemaphoreType.DMA,
            pltpu.SemaphoreType.DMA((N_DEV - 1,)),
            pltpu.SemaphoreType.DMA((N_DEV,)),
        ],
        compiler_params=pltpu.CompilerParams(collective_id=0),
    )(kvp, xb, wqb, wob)
    return out


def kernel(x, Wq, K_ext, V_ext, Wo):
    xb = x[0].astype(jnp.bfloat16)
    wqb = Wq.astype(jnp.bfloat16)
    wob = Wo.astype(jnp.bfloat16)
    kvp = jnp.concatenate(
        [K_ext[0].transpose(1, 0, 2).astype(jnp.bfloat16),
         V_ext[0].transpose(1, 0, 2).astype(jnp.bfloat16)],
        axis=-1,
    )
    return _fused(kvp, xb, wqb, wob)


# device time: 359687 ns/iter; 1.1222x vs baseline; 1.1222x over previous
import jax
import jax.numpy as jnp
from jax import lax
from jax.experimental import pallas as pl
from jax.experimental.pallas import tpu as pltpu

N_DEV = 4
S = 2048
H = 8
DH = 128
D = 1024
BLK = 64
SCALE = 0.08838834764831843
NEG = -1e9
QT = 256
N_QT = S // QT
N_HQ = H * N_QT


def _fused(kvp, xb, wqb, wob):
    def body(kvp_ref, xb_ref, wq_ref, wo_ref, out_ref,
             kvall_ref, q_ref, kvbuf, acc_ref, m_ref, l_ref, ctx_ref,
             chunk_sem, send_sems, recv_sems):
        my = lax.axis_index("i")

        barrier_sem = pltpu.get_barrier_semaphore()
        for off in range(1, N_DEV):
            pl.semaphore_signal(
                barrier_sem, inc=1,
                device_id=(lax.rem(my + off, N_DEV),),
                device_id_type=pl.DeviceIdType.MESH,
            )
        pl.semaphore_wait(barrier_sem, N_DEV - 1)

        for off in range(1, N_DEV):
            @pl.when(my + off <= N_DEV - 1)
            def _send():
                rdma = pltpu.make_async_remote_copy(
                    src_ref=kvp_ref,
                    dst_ref=kvall_ref.at[my],
                    send_sem=send_sems.at[off - 1],
                    recv_sem=recv_sems.at[my],
                    device_id=(my + off,),
                    device_id_type=pl.DeviceIdType.MESH,
                )
                rdma.start()

        for g in range(H // 2):
            qp = jnp.dot(xb_ref[...], wq_ref[:, g * 2 * DH:(g + 1) * 2 * DH],
                         preferred_element_type=jnp.float32) * SCALE
            q_ref[2 * g] = qp[:, 0:DH].astype(jnp.bfloat16)
            q_ref[2 * g + 1] = qp[:, DH:2 * DH].astype(jnp.bfloat16)
        m_ref[...] = jnp.full((N_HQ, 1, QT), -1e30, jnp.float32)
        l_ref[...] = jnp.zeros((N_HQ, 1, QT), jnp.float32)
        acc_ref[...] = jnp.zeros((N_HQ, DH, QT), jnp.float32)

        def flash_update(idx, vh, sT):
            m_prev = m_ref[idx]
            m_new = jnp.maximum(
                m_prev, jnp.max(sT, axis=0, keepdims=True))
            alpha = jnp.exp(m_prev - m_new)
            p = jnp.exp(sT - m_new)
            l_ref[idx] = l_ref[idx] * alpha + jnp.sum(
                p, axis=0, keepdims=True)
            acc_ref[idx] = acc_ref[idx] * alpha + lax.dot_general(
                vh, p.astype(jnp.bfloat16), (((0,), (0,)), ((), ())),
                preferred_element_type=jnp.float32,
            )
            m_ref[idx] = m_new

        def qk_tile(idx):
            h = idx // N_QT
            qt = idx % N_QT
            qh = q_ref[h, pl.ds(qt * QT, QT), :]
            kh = kvbuf[h, :, 0:DH]
            return lax.dot_general(
                kh, qh, (((1,), (1,)), ((), ())),
                preferred_element_type=jnp.float32,
            ).astype(jnp.bfloat16)

        def run_full_chunk():
            def hq_body(j, dummy):
                ia = 2 * j
                ib = 2 * j + 1
                sa = qk_tile(ia)
                sb = qk_tile(ib)
                va = kvbuf[ia // N_QT, :, DH:2 * DH]
                flash_update(ia, va, sa.astype(jnp.float32))
                vb = kvbuf[ib // N_QT, :, DH:2 * DH]
                flash_update(ib, vb, sb.astype(jnp.float32))
                return dummy

            lax.fori_loop(0, N_HQ // 2, hq_body, 0)

        def make_diag_body(qt):
            rs = qt * QT
            kl = rs + QT

            def diag_body(h, dummy):
                qh = q_ref[h, rs:rs + QT, :]
                kh = kvbuf[h, 0:kl, 0:DH]
                vh = kvbuf[h, 0:kl, DH:2 * DH]
                sT = lax.dot_general(
                    kh, qh, (((1,), (1,)), ((), ())),
                    preferred_element_type=jnp.float32,
                )
                kb = lax.broadcasted_iota(jnp.int32, (kl, 1), 0) // BLK
                qb = (rs
                      + lax.broadcasted_iota(jnp.int32, (1, QT), 1)
                      ) // BLK
                sT = jnp.where(kb <= qb, sT, NEG)
                flash_update(h * N_QT + qt, vh, sT)
                return dummy
            return diag_body

        for k in range(N_DEV):
            c = my - k

            @pl.when(c >= 0)
            def _chunk():
                if k == 0:
                    cp = pltpu.make_async_copy(kvp_ref, kvbuf, chunk_sem)
                    cp.start()
                    cp.wait()
                    for qt in range(N_QT):
                        lax.fori_loop(0, H, make_diag_body(qt), 0)
                else:
                    pltpu.make_async_remote_copy(
                        src_ref=kvall_ref.at[c],
                        dst_ref=kvall_ref.at[c],
                        send_sem=send_sems.at[0],
                        recv_sem=recv_sems.at[c],
                        device_id=(my,),
                        device_id_type=pl.DeviceIdType.MESH,
                    ).wait_recv()
                    cp = pltpu.make_async_copy(
                        kvall_ref.at[c], kvbuf, chunk_sem)
                    cp.start()
                    cp.wait()
                    run_full_chunk()

        def ep_body(idx, dummy):
            h = idx // N_QT
            qt = idx % N_QT
            rs = qt * QT
            ctx_ref[pl.ds(h * DH, DH), pl.ds(rs, QT)] = (
                acc_ref[idx] / l_ref[idx]).astype(jnp.bfloat16)
            return dummy

        lax.fori_loop(0, N_HQ, ep_body, 0)
        for rb in range(4):
            rs = rb * (S // 4)
            out_ref[0, rs:rs + S // 4, :] = lax.dot_general(
                ctx_ref[:, rs:rs + S // 4], wo_ref[...],
                (((0,), (0,)), ((), ())),
                preferred_element_type=jnp.float32,
            )

        for off in range(1, N_DEV):
            @pl.when(my + off <= N_DEV - 1)
            def _drain():
                pltpu.make_async_remote_copy(
                    src_ref=kvp_ref,
                    dst_ref=kvall_ref.at[my],
                    send_sem=send_sems.at[off - 1],
                    recv_sem=recv_sems.at[0],
                    device_id=(my,),
                    device_id_type=pl.DeviceIdType.MESH,
                ).wait_send()

    out, _ = pl.pallas_call(
        body,
        out_shape=[
            jax.ShapeDtypeStruct((1, S, D), jnp.float32),
            jax.ShapeDtypeStruct((N_DEV, H, S, 2 * DH), jnp.bfloat16),
        ],
        in_specs=[
            pl.BlockSpec(memory_space=pltpu.MemorySpace.HBM),
            pl.BlockSpec(memory_space=pltpu.MemorySpace.VMEM),
            pl.BlockSpec(memory_space=pltpu.MemorySpace.VMEM),
            pl.BlockSpec(memory_space=pltpu.MemorySpace.VMEM),
        ],
        out_specs=[
            pl.BlockSpec(memory_space=pltpu.MemorySpace.VMEM),
            pl.BlockSpec(memory_space=pltpu.MemorySpace.HBM),
        ],
        scratch_shapes=[
            pltpu.VMEM((H, S, DH), jnp.bfloat16),
            pltpu.VMEM((H, S, 2 * DH), jnp.bfloat16),
            pltpu.VMEM((N_HQ, DH, QT), jnp.float32),
            pltpu.VMEM((N_HQ, 1, QT), jnp.float32),
            pltpu.VMEM((N_HQ, 1, QT), jnp.float32),
            pltpu.VMEM((H * DH, S), jnp.bfloat16),
            pltpu.SemaphoreType.DMA,
            pltpu.SemaphoreType.DMA((N_DEV - 1,)),
            pltpu.SemaphoreType.DMA((N_DEV,)),
        ],
        compiler_params=pltpu.CompilerParams(collective_id=0),
    )(kvp, xb, wqb, wob)
    return out


def kernel(x, Wq, K_ext, V_ext, Wo):
    xb = x[0].astype(jnp.bfloat16)
    wqb = Wq.astype(jnp.bfloat16)
    wob = Wo.astype(jnp.bfloat16)
    kvp = jnp.concatenate(
        [K_ext[0].transpose(1, 0, 2).astype(jnp.bfloat16),
         V_ext[0].transpose(1, 0, 2).astype(jnp.bfloat16)],
        axis=-1,
    )
    return _fused(kvp, xb, wqb, wob)


# device time: 358803 ns/iter; 1.1250x vs baseline; 1.0025x over previous
import jax
import jax.numpy as jnp
from jax import lax
from jax.experimental import pallas as pl
from jax.experimental.pallas import tpu as pltpu

N_DEV = 4
S = 2048
H = 8
DH = 128
D = 1024
BLK = 64
SCALE = 0.08838834764831843
NEG = -1e9
QT = 256
N_QT = S // QT
N_HQ = H * N_QT


def _fused(kvp, xb, wqb, wob):
    def body(kvp_ref, xb_ref, wq_ref, wo_ref, out_ref,
             kvall_ref, q_ref, kvbuf, acc_ref, m_ref, l_ref, ctx_ref,
             chunk_sem, send_sems, recv_sems):
        my = lax.axis_index("i")

        barrier_sem = pltpu.get_barrier_semaphore()
        for off in range(1, N_DEV):
            pl.semaphore_signal(
                barrier_sem, inc=1,
                device_id=(lax.rem(my + off, N_DEV),),
                device_id_type=pl.DeviceIdType.MESH,
            )
        pl.semaphore_wait(barrier_sem, N_DEV - 1)

        for off in range(1, N_DEV):
            @pl.when(my + off <= N_DEV - 1)
            def _send():
                rdma = pltpu.make_async_remote_copy(
                    src_ref=kvp_ref,
                    dst_ref=kvall_ref.at[my],
                    send_sem=send_sems.at[off - 1],
                    recv_sem=recv_sems.at[my],
                    device_id=(my + off,),
                    device_id_type=pl.DeviceIdType.MESH,
                )
                rdma.start()

        for g in range(H // 2):
            qp = jnp.dot(xb_ref[...], wq_ref[:, g * 2 * DH:(g + 1) * 2 * DH],
                         preferred_element_type=jnp.float32) * SCALE
            q_ref[2 * g] = qp[:, 0:DH].astype(jnp.bfloat16)
            q_ref[2 * g + 1] = qp[:, DH:2 * DH].astype(jnp.bfloat16)
        m_ref[...] = jnp.full((N_HQ, 1, QT), -1e30, jnp.float32)
        l_ref[...] = jnp.zeros((N_HQ, 1, QT), jnp.float32)
        acc_ref[...] = jnp.zeros((N_HQ, DH, QT), jnp.float32)

        def flash_update(idx, vh, sT):
            m_prev = m_ref[idx]
            m_new = jnp.maximum(
                m_prev, jnp.max(sT, axis=0, keepdims=True))
            alpha = jnp.exp(m_prev - m_new)
            p = jnp.exp(sT - m_new)
            l_ref[idx] = l_ref[idx] * alpha + jnp.sum(
                p, axis=0, keepdims=True)
            acc_ref[idx] = acc_ref[idx] * alpha + lax.dot_general(
                vh, p.astype(jnp.bfloat16), (((0,), (0,)), ((), ())),
                preferred_element_type=jnp.float32,
            )
            m_ref[idx] = m_new

        def qk_tile(idx):
            h = idx // N_QT
            qt = idx % N_QT
            qh = q_ref[h, pl.ds(qt * QT, QT), :]
            kh = kvbuf[h, :, 0:DH]
            return lax.dot_general(
                kh, qh, (((1,), (1,)), ((), ())),
                preferred_element_type=jnp.float32,
            ).astype(jnp.bfloat16)

        def run_full_chunk():
            def hq_body(j, dummy):
                ia = 2 * j
                ib = 2 * j + 1
                sa = qk_tile(ia)
                sb = qk_tile(ib)
                va = kvbuf[ia // N_QT, :, DH:2 * DH]
                flash_update(ia, va, sa.astype(jnp.float32))
                vb = kvbuf[ib // N_QT, :, DH:2 * DH]
                flash_update(ib, vb, sb.astype(jnp.float32))
                return dummy

            lax.fori_loop(0, N_HQ // 2, hq_body, 0)

        def make_diag_body(qt):
            rs = qt * QT
            kl = rs + QT

            def scores(h):
                qh = q_ref[h, rs:rs + QT, :]
                kh = kvbuf[h, 0:kl, 0:DH]
                return lax.dot_general(
                    kh, qh, (((1,), (1,)), ((), ())),
                    preferred_element_type=jnp.float32,
                )

            def diag_body(j, dummy):
                ha = 2 * j
                hb = 2 * j + 1
                sa = scores(ha)
                sb = scores(hb)
                kb = lax.broadcasted_iota(jnp.int32, (kl, 1), 0) // BLK
                qb = (rs
                      + lax.broadcasted_iota(jnp.int32, (1, QT), 1)
                      ) // BLK
                mask = kb <= qb
                va = kvbuf[ha, 0:kl, DH:2 * DH]
                flash_update(ha * N_QT + qt, va, jnp.where(mask, sa, NEG))
                vb = kvbuf[hb, 0:kl, DH:2 * DH]
                flash_update(hb * N_QT + qt, vb, jnp.where(mask, sb, NEG))
                return dummy
            return diag_body

        for k in range(N_DEV):
            c = my - k

            @pl.when(c >= 0)
            def _chunk():
                if k == 0:
                    cp = pltpu.make_async_copy(kvp_ref, kvbuf, chunk_sem)
                    cp.start()
                    cp.wait()
                    for qt in range(N_QT):
                        lax.fori_loop(0, H // 2, make_diag_body(qt), 0)
                else:
                    pltpu.make_async_remote_copy(
                        src_ref=kvall_ref.at[c],
                        dst_ref=kvall_ref.at[c],
                        send_sem=send_sems.at[0],
                        recv_sem=recv_sems.at[c],
                        device_id=(my,),
                        device_id_type=pl.DeviceIdType.MESH,
                    ).wait_recv()
                    cp = pltpu.make_async_copy(
                        kvall_ref.at[c], kvbuf, chunk_sem)
                    cp.start()
                    cp.wait()
                    run_full_chunk()

        def ep_body(idx, dummy):
            h = idx // N_QT
            qt = idx % N_QT
            rs = qt * QT
            ctx_ref[pl.ds(h * DH, DH), pl.ds(rs, QT)] = (
                acc_ref[idx] / l_ref[idx]).astype(jnp.bfloat16)
            return dummy

        lax.fori_loop(0, N_HQ, ep_body, 0)
        for rb in range(4):
            rs = rb * (S // 4)
            out_ref[0, rs:rs + S // 4, :] = lax.dot_general(
                ctx_ref[:, rs:rs + S // 4], wo_ref[...],
                (((0,), (0,)), ((), ())),
                preferred_element_type=jnp.float32,
            )

        for off in range(1, N_DEV):
            @pl.when(my + off <= N_DEV - 1)
            def _drain():
                pltpu.make_async_remote_copy(
                    src_ref=kvp_ref,
                    dst_ref=kvall_ref.at[my],
                    send_sem=send_sems.at[off - 1],
                    recv_sem=recv_sems.at[0],
                    device_id=(my,),
                    device_id_type=pl.DeviceIdType.MESH,
                ).wait_send()

    out, _ = pl.pallas_call(
        body,
        out_shape=[
            jax.ShapeDtypeStruct((1, S, D), jnp.float32),
            jax.ShapeDtypeStruct((N_DEV, H, S, 2 * DH), jnp.bfloat16),
        ],
        in_specs=[
            pl.BlockSpec(memory_space=pltpu.MemorySpace.HBM),
            pl.BlockSpec(memory_space=pltpu.MemorySpace.VMEM),
            pl.BlockSpec(memory_space=pltpu.MemorySpace.VMEM),
            pl.BlockSpec(memory_space=pltpu.MemorySpace.VMEM),
        ],
        out_specs=[
            pl.BlockSpec(memory_space=pltpu.MemorySpace.VMEM),
            pl.BlockSpec(memory_space=pltpu.MemorySpace.HBM),
        ],
        scratch_shapes=[
            pltpu.VMEM((H, S, DH), jnp.bfloat16),
            pltpu.VMEM((H, S, 2 * DH), jnp.bfloat16),
            pltpu.VMEM((N_HQ, DH, QT), jnp.float32),
            pltpu.VMEM((N_HQ, 1, QT), jnp.float32),
            pltpu.VMEM((N_HQ, 1, QT), jnp.float32),
            pltpu.VMEM((H * DH, S), jnp.bfloat16),
            pltpu.SemaphoreType.DMA,
            pltpu.SemaphoreType.DMA((N_DEV - 1,)),
            pltpu.SemaphoreType.DMA((N_DEV,)),
        ],
        compiler_params=pltpu.CompilerParams(collective_id=0),
    )(kvp, xb, wqb, wob)
    return out


def kernel(x, Wq, K_ext, V_ext, Wo):
    xb = x[0].astype(jnp.bfloat16)
    wqb = Wq.astype(jnp.bfloat16)
    wob = Wo.astype(jnp.bfloat16)
    kvp = jnp.concatenate(
        [K_ext[0].transpose(1, 0, 2).astype(jnp.bfloat16),
         V_ext[0].transpose(1, 0, 2).astype(jnp.bfloat16)],
        axis=-1,
    )
    return _fused(kvp, xb, wqb, wob)
